# baseline (speedup 1.0000x reference)
import functools

import jax
import jax.numpy as jnp
from jax import lax
from jax.experimental import pallas as pl
from jax.experimental.pallas import tpu as pltpu
from jax.experimental.pallas import tpu_sc as plsc

N = 10000
E = 320000
NUM_HE = 10000
HID = 128
NCLS = 10
NGRAPH = 64

NC, NS = 2, 16
NW = NC * NS
CH = 112
SB = 10
NBLK = 9
EPT = NBLK * SB * CH
EPAD = NW * EPT
RPAD = 10240
RPT = RPAD // NS

_f32 = jnp.float32


@functools.cache
def _spmm_sc():
    mesh = plsc.VectorSubcoreMesh(core_axis_name="c", subcore_axis_name="s")

    @functools.partial(
        pl.kernel,
        out_type=jax.ShapeDtypeStruct((NC, RPAD, HID), _f32),
        mesh=mesh,
        scratch_types=[
            pltpu.VMEM((2, SB, CH), jnp.int32),
            pltpu.VMEM((2, SB, CH), jnp.int32),
            pltpu.VMEM((CH, HID), _f32),
            pltpu.VMEM((CH, HID), _f32),
            pltpu.VMEM((16, HID), _f32),
            pltpu.VMEM_SHARED((RPAD, HID), _f32),
            pltpu.SemaphoreType.DMA,
            pltpu.SemaphoreType.DMA,
            pltpu.SemaphoreType.DMA,
            pltpu.SemaphoreType.DMA,
            pltpu.SemaphoreType.DMA,
            pltpu.SemaphoreType.DMA,
        ],
    )
    def spmm(x_hbm, src_hbm, dst_hbm, out_hbm,
             sidx, didx, rows0, rows1, zbuf, ysh,
             sem0, sem1, sema, semb, semi, semj):
        c = lax.axis_index("c")
        s = lax.axis_index("s")
        wid = c * NS + s
        src_t = src_hbm.at[wid]
        dst_t = dst_hbm.at[wid]

        @pl.loop(0, 16)
        def _(i):
            @pl.loop(0, HID, step=16)
            def _(j):
                zbuf[i, pl.ds(j, 16)] = jnp.zeros((16,), _f32)

        @pl.loop(0, RPT, step=16)
        def _(k):
            pltpu.sync_copy(zbuf, ysh.at[pl.ds(s * RPT + k, 16)])

        plsc.subcore_barrier()

        def idx_load(b, slot):
            return (pltpu.make_async_copy(src_t.at[b], sidx.at[slot], semi),
                    pltpu.make_async_copy(dst_t.at[b], didx.at[slot], semj))

        for cp in idx_load(0, 0):
            cp.start()
            cp.wait()

        for b in range(NBLK):
            slot = b % 2
            if b > 0:
                for cp in idx_load(b, slot):
                    cp.wait()
            if b + 1 < NBLK:
                for cp in idx_load(b + 1, (b + 1) % 2):
                    cp.start()
            sb = sidx.at[slot]
            db = didx.at[slot]
            pltpu.make_async_copy(x_hbm.at[sb.at[0]], rows0, sem0).start()
            pltpu.make_async_copy(x_hbm.at[sb.at[1]], rows1, sem1).start()

            @pl.loop(0, SB - 2, step=2)
            def _(j):
                pltpu.make_async_copy(x_hbm.at[sb.at[j]], rows0, sem0).wait()
                sc0 = pltpu.async_copy(rows0, ysh.at[db.at[j]], sema, add=True)
                pltpu.make_async_copy(x_hbm.at[sb.at[j + 1]], rows1, sem1).wait()
                sc1 = pltpu.async_copy(rows1, ysh.at[db.at[j + 1]], semb, add=True)
                sc0.wait()
                pltpu.make_async_copy(x_hbm.at[sb.at[j + 2]], rows0, sem0).start()
                sc1.wait()
                pltpu.make_async_copy(x_hbm.at[sb.at[j + 3]], rows1, sem1).start()

            pltpu.make_async_copy(x_hbm.at[sb.at[SB - 2]], rows0, sem0).wait()
            sc0 = pltpu.async_copy(rows0, ysh.at[db.at[SB - 2]], sema, add=True)
            pltpu.make_async_copy(x_hbm.at[sb.at[SB - 1]], rows1, sem1).wait()
            sc1 = pltpu.async_copy(rows1, ysh.at[db.at[SB - 1]], semb, add=True)
            sc0.wait()
            sc1.wait()

        plsc.subcore_barrier()
        pltpu.sync_copy(ysh.at[pl.ds(s * RPT, RPT)],
                        out_hbm.at[c].at[pl.ds(s * RPT, RPT)])

    return spmm


@functools.cache
def _deg_sc():
    mesh = plsc.VectorSubcoreMesh(core_axis_name="c", subcore_axis_name="s")

    @functools.partial(
        pl.kernel,
        out_type=(jax.ShapeDtypeStruct((NC, RPAD), _f32),
                  jax.ShapeDtypeStruct((NC, RPAD), _f32)),
        mesh=mesh,
        scratch_types=[
            pltpu.VMEM((NBLK, SB, CH), jnp.int32),
            pltpu.VMEM((NBLK, SB, CH), jnp.int32),
            pltpu.VMEM((CH,), _f32),
            pltpu.VMEM((RPT,), _f32),
            pltpu.VMEM_SHARED((RPAD,), _f32),
            pltpu.VMEM_SHARED((RPAD,), _f32),
        ],
    )
    def deg(src_hbm, dst_hbm, outn_hbm, outh_hbm,
            sidx, didx, ones, zb, dn_sh, dh_sh):
        c = lax.axis_index("c")
        s = lax.axis_index("s")
        wid = c * NS + s

        pltpu.sync_copy(src_hbm.at[wid], sidx)
        pltpu.sync_copy(dst_hbm.at[wid], didx)

        @pl.loop(0, CH, step=16)
        def _(i):
            ones[pl.ds(i, 16)] = jnp.ones((16,), _f32)

        @pl.loop(0, RPT, step=16)
        def _(i):
            zb[pl.ds(i, 16)] = jnp.zeros((16,), _f32)

        pltpu.sync_copy(zb, dn_sh.at[pl.ds(s * RPT, RPT)])
        pltpu.sync_copy(zb, dh_sh.at[pl.ds(s * RPT, RPT)])
        plsc.subcore_barrier()

        @pl.loop(0, NBLK)
        def _(b):
            @pl.loop(0, SB)
            def _(j):
                pltpu.sync_copy(ones, dn_sh.at[sidx.at[b].at[j]], add=True)
                pltpu.sync_copy(ones, dh_sh.at[didx.at[b].at[j]], add=True)

        plsc.subcore_barrier()
        pltpu.sync_copy(dn_sh.at[pl.ds(s * RPT, RPT)],
                        outn_hbm.at[c].at[pl.ds(s * RPT, RPT)])
        pltpu.sync_copy(dh_sh.at[pl.ds(s * RPT, RPT)],
                        outh_hbm.at[c].at[pl.ds(s * RPT, RPT)])

    return deg


def _spmm(x, src3, dst3):
    return _spmm_sc()(x, src3, dst3)


def _deg(src3, dst3):
    return _deg_sc()(src3, dst3)


_RB = 2000


def _dot(a, b):
    return lax.dot_general(a, b, (((1,), (0,)), ((), ())),
                           precision=lax.Precision.HIGHEST,
                           preferred_element_type=_f32)


def _mm(x, w):
    def body(x_ref, w_ref, o_ref):
        o_ref[...] = _dot(x_ref[...], w_ref[...])

    return pl.pallas_call(
        body,
        grid=(N // _RB,),
        in_specs=[pl.BlockSpec((_RB, HID), lambda i: (i, 0)),
                  pl.BlockSpec((HID, HID), lambda i: (0, 0))],
        out_specs=pl.BlockSpec((_RB, HID), lambda i: (i, 0)),
        out_shape=jax.ShapeDtypeStruct((N, HID), _f32),
    )(x, w)


def _inv(dn0, dn1, dh0, dh1):
    def body(a_ref, b_ref, c_ref, d_ref, o1_ref, o2_ref):
        dn = a_ref[...] + b_ref[...]
        ri = lax.broadcasted_iota(jnp.int32, (N, 1), 0)
        dn = jnp.where(ri == 0, dn - float(EPAD - E), dn)
        dh = c_ref[...] + d_ref[...]
        o1_ref[...] = jnp.where(dn > 0, 1.0 / jnp.where(dn > 0, dn, 1.0), 0.0)
        o2_ref[...] = jnp.where(dh > 0, 1.0 / jnp.where(dh > 0, dh, 1.0), 0.0)

    return pl.pallas_call(
        body,
        grid=(1,),
        in_specs=[pl.BlockSpec((N, 1), lambda i: (0, 0))] * 4,
        out_specs=[pl.BlockSpec((N, 1), lambda i: (0, 0))] * 2,
        out_shape=[jax.ShapeDtypeStruct((N, 1), _f32)] * 2,
    )(dn0, dn1, dh0, dh1)


def _pp(i):
    return (0, i, 0)


def _pq(i):
    return (1, i, 0)


_PSPEC = [pl.BlockSpec((1, _RB, HID), _pp), pl.BlockSpec((1, _RB, HID), _pq)]


def _mid(p, binv):
    def body(p0_ref, p1_ref, s_ref, o_ref):
        o_ref[...] = s_ref[...] * (p0_ref[0] + p1_ref[0])

    return pl.pallas_call(
        body,
        grid=(N // _RB,),
        in_specs=_PSPEC + [pl.BlockSpec((_RB, 1), lambda i: (i, 0))],
        out_specs=pl.BlockSpec((_RB, HID), lambda i: (i, 0)),
        out_shape=jax.ShapeDtypeStruct((N, HID), _f32),
    )(p, p, binv)


def _endmm(q, dinv, b, w):
    def body(q0_ref, q1_ref, s_ref, b_ref, w_ref, o_ref):
        h = jnp.maximum(s_ref[...] * (q0_ref[0] + q1_ref[0]) + b_ref[...], 0.0)
        o_ref[...] = _dot(h, w_ref[...])

    return pl.pallas_call(
        body,
        grid=(N // _RB,),
        in_specs=_PSPEC + [pl.BlockSpec((_RB, 1), lambda i: (i, 0)),
                           pl.BlockSpec((1, HID), lambda i: (0, 0)),
                           pl.BlockSpec((HID, HID), lambda i: (0, 0))],
        out_specs=pl.BlockSpec((_RB, HID), lambda i: (i, 0)),
        out_shape=jax.ShapeDtypeStruct((N, HID), _f32),
    )(q, q, dinv, b, w)


def _end(q, dinv, b):
    def body(q0_ref, q1_ref, s_ref, b_ref, o_ref):
        o_ref[...] = jnp.maximum(
            s_ref[...] * (q0_ref[0] + q1_ref[0]) + b_ref[...], 0.0)

    return pl.pallas_call(
        body,
        grid=(N // _RB,),
        in_specs=_PSPEC + [pl.BlockSpec((_RB, 1), lambda i: (i, 0)),
                           pl.BlockSpec((1, HID), lambda i: (0, 0))],
        out_specs=pl.BlockSpec((_RB, HID), lambda i: (i, 0)),
        out_shape=jax.ShapeDtypeStruct((N, HID), _f32),
    )(q, q, dinv, b)


_NEG = -1e30


def _pool(h3, bm, g1w, g1b, g2w, g2b, wc, bc):
    nblk = N // _RB

    def body(h_ref, bm_ref, g1w_ref, g1b_ref, g2w_ref, g2b_ref,
             wc_ref, bc_ref, o_ref, gmax_s, num_s, den_s):
        p = pl.program_id(0)
        i = pl.program_id(1)

        @pl.when(jnp.logical_and(p == 0, i == 0))
        def _():
            gmax_s[...] = jnp.full((1, NGRAPH), _NEG, _f32)
            num_s[...] = jnp.zeros((NGRAPH, HID), _f32)
            den_s[...] = jnp.zeros((NGRAPH, 1), _f32)

        ids = lax.broadcasted_iota(jnp.int32, (1, NGRAPH), 1)
        mask = bm_ref[...] == ids
        maskf = mask.astype(_f32)
        z = jnp.maximum(_dot(h_ref[...], g1w_ref[...]) + g1b_ref[...], 0.0)
        gate = _dot(z, g2w_ref[...]) + g2b_ref[...]

        @pl.when(p == 0)
        def _():
            gm = jnp.max(jnp.where(mask, gate, _NEG), axis=0, keepdims=True)
            gmax_s[...] = jnp.maximum(gmax_s[...], gm)

        @pl.when(p == 1)
        def _():
            gmax_node = jnp.sum(jnp.where(mask, gmax_s[...], 0.0),
                                axis=1, keepdims=True)
            egate = jnp.exp(gate - gmax_node)
            den_s[...] += lax.dot_general(
                maskf, egate, (((0,), (0,)), ((), ())),
                precision=lax.Precision.HIGHEST, preferred_element_type=_f32)
            num_s[...] += lax.dot_general(
                maskf, egate * h_ref[...], (((0,), (0,)), ((), ())),
                precision=lax.Precision.HIGHEST, preferred_element_type=_f32)

            @pl.when(i == nblk - 1)
            def _():
                pooled = num_s[...] * (1.0 / (den_s[...] + 1e-16))
                o_ref[...] = _dot(pooled, wc_ref[...]) + bc_ref[...]

    return pl.pallas_call(
        body,
        grid=(2, nblk),
        in_specs=[pl.BlockSpec((_RB, HID), lambda p, i: (i, 0)),
                  pl.BlockSpec((_RB, 1), lambda p, i: (i, 0)),
                  pl.BlockSpec((HID, HID), lambda p, i: (0, 0)),
                  pl.BlockSpec((1, HID), lambda p, i: (0, 0)),
                  pl.BlockSpec((HID, 1), lambda p, i: (0, 0)),
                  pl.BlockSpec((1, 1), lambda p, i: (0, 0)),
                  pl.BlockSpec((HID, NCLS), lambda p, i: (0, 0)),
                  pl.BlockSpec((1, NCLS), lambda p, i: (0, 0))],
        out_specs=pl.BlockSpec((NGRAPH, NCLS), lambda p, i: (0, 0)),
        out_shape=jax.ShapeDtypeStruct((NGRAPH, NCLS), _f32),
        scratch_shapes=[pltpu.VMEM((1, NGRAPH), _f32),
                        pltpu.VMEM((NGRAPH, HID), _f32),
                        pltpu.VMEM((NGRAPH, 1), _f32)],
    )(h3, bm, g1w, g1b, g2w, g2b, wc, bc)


def kernel(x, edge_index, batch_map, W1, b1, W2, b2, W3, b3,
           G1, g1, G2, g2, Wc, bc):
    npad = EPAD - E
    zpad = jnp.zeros((npad,), jnp.int32)
    gpad = jnp.full((npad,), N, jnp.int32)
    sh4 = (NW, NBLK, SB, CH)
    src3 = jnp.concatenate([edge_index[0], zpad]).reshape(sh4)
    dst3 = jnp.concatenate([edge_index[1], gpad]).reshape(sh4)
    rsrc3 = jnp.concatenate([edge_index[1], zpad]).reshape(sh4)
    rdst3 = jnp.concatenate([edge_index[0], gpad]).reshape(sh4)

    degn, degh = _deg(src3, dst3)
    dinv, binv = _inv(degn[0, :N, None], degn[1, :N, None],
                      degh[0, :N, None], degh[1, :N, None])

    h = _mm(x, W1)
    for b, wnext in ((b1, W2), (b2, W3)):
        m = _mid(_spmm(h, src3, dst3), binv)
        h = _endmm(_spmm(m, rsrc3, rdst3), dinv, b[None, :], wnext)
    m = _mid(_spmm(h, src3, dst3), binv)
    h3 = _end(_spmm(m, rsrc3, rdst3), dinv, b3[None, :])

    return _pool(h3, batch_map[:, None], G1, g1[None, :], G2,
                 g2.reshape(1, 1), Wc, bc[None, :])


# speedup vs baseline: 1.5740x; 1.5740x over previous
import functools

import jax
import jax.numpy as jnp
from jax import lax
from jax.experimental import pallas as pl
from jax.experimental.pallas import tpu as pltpu
from jax.experimental.pallas import tpu_sc as plsc

N = 10000
E = 320000
NUM_HE = 10000
HID = 128
NCLS = 10
NGRAPH = 64

NC, NS = 2, 16
NW = NC * NS
CH = 80
SB = 25
NBLK = 5
EPT = NBLK * SB * CH
EPAD = NW * EPT
RPAD = 10240
RPT = RPAD // NS

_f32 = jnp.float32


@functools.cache
def _spmm_sc():
    mesh = plsc.VectorSubcoreMesh(core_axis_name="c", subcore_axis_name="s")

    @functools.partial(
        pl.kernel,
        out_type=jax.ShapeDtypeStruct((NC, RPAD, HID), _f32),
        mesh=mesh,
        scratch_types=[
            pltpu.VMEM((2, SB, CH), jnp.int32),
            pltpu.VMEM((2, SB, CH), jnp.int32),
            pltpu.VMEM((CH, HID), _f32),
            pltpu.VMEM((CH, HID), _f32),
            pltpu.VMEM((16, HID), _f32),
            pltpu.VMEM_SHARED((RPAD, HID), _f32),
            pltpu.SemaphoreType.DMA,
            pltpu.SemaphoreType.DMA,
            pltpu.SemaphoreType.DMA,
            pltpu.SemaphoreType.DMA,
            pltpu.SemaphoreType.DMA,
            pltpu.SemaphoreType.DMA,
        ],
    )
    def spmm(x_hbm, src_hbm, dst_hbm, out_hbm,
             sidx, didx, rows0, rows1, zbuf, ysh,
             sem0, sem1, sema, semb, semi, semj):
        c = lax.axis_index("c")
        s = lax.axis_index("s")
        wid = c * NS + s
        src_t = src_hbm.at[wid]
        dst_t = dst_hbm.at[wid]

        @pl.loop(0, 16)
        def _(i):
            @pl.loop(0, HID, step=16)
            def _(j):
                zbuf[i, pl.ds(j, 16)] = jnp.zeros((16,), _f32)

        @pl.loop(0, RPT, step=16)
        def _(k):
            pltpu.sync_copy(zbuf, ysh.at[pl.ds(s * RPT + k, 16)])

        plsc.subcore_barrier()

        def idx_load(b, slot):
            return (pltpu.make_async_copy(src_t.at[b], sidx.at[slot], semi),
                    pltpu.make_async_copy(dst_t.at[b], didx.at[slot], semj))

        for cp in idx_load(0, 0):
            cp.start()
            cp.wait()

        for b in range(NBLK):
            slot = b % 2
            if b > 0:
                for cp in idx_load(b, slot):
                    cp.wait()
            if b + 1 < NBLK:
                for cp in idx_load(b + 1, (b + 1) % 2):
                    cp.start()
            sb = sidx.at[slot]
            db = didx.at[slot]
            pltpu.make_async_copy(x_hbm.at[sb.at[0]], rows0, sem0).start()
            pltpu.make_async_copy(x_hbm.at[sb.at[1]], rows1, sem1).start()

            @pl.loop(0, SB - 1, step=2)
            def _(j):
                pltpu.make_async_copy(x_hbm.at[sb.at[j]], rows0, sem0).wait()
                sc0 = pltpu.async_copy(rows0, ysh.at[db.at[j]], sema, add=True)
                pltpu.make_async_copy(x_hbm.at[sb.at[j + 1]], rows1, sem1).wait()
                sc1 = pltpu.async_copy(rows1, ysh.at[db.at[j + 1]], semb, add=True)
                sc0.wait()
                pltpu.make_async_copy(x_hbm.at[sb.at[j + 2]], rows0, sem0).start()
                sc1.wait()

                @pl.when(j + 3 < SB)
                def _():
                    pltpu.make_async_copy(x_hbm.at[sb.at[j + 3]], rows1, sem1).start()

            pltpu.make_async_copy(x_hbm.at[sb.at[SB - 1]], rows0, sem0).wait()
            pltpu.sync_copy(rows0, ysh.at[db.at[SB - 1]], add=True)

        plsc.subcore_barrier()
        pltpu.sync_copy(ysh.at[pl.ds(s * RPT, RPT)],
                        out_hbm.at[c].at[pl.ds(s * RPT, RPT)])

    return spmm


@functools.cache
def _deg_sc():
    mesh = plsc.VectorSubcoreMesh(core_axis_name="c", subcore_axis_name="s")

    @functools.partial(
        pl.kernel,
        out_type=(jax.ShapeDtypeStruct((NC, RPAD), _f32),
                  jax.ShapeDtypeStruct((NC, RPAD), _f32)),
        mesh=mesh,
        scratch_types=[
            pltpu.VMEM((NBLK, SB, CH), jnp.int32),
            pltpu.VMEM((NBLK, SB, CH), jnp.int32),
            pltpu.VMEM((CH,), _f32),
            pltpu.VMEM((RPT,), _f32),
            pltpu.VMEM_SHARED((RPAD,), _f32),
            pltpu.VMEM_SHARED((RPAD,), _f32),
        ],
    )
    def deg(src_hbm, dst_hbm, outn_hbm, outh_hbm,
            sidx, didx, ones, zb, dn_sh, dh_sh):
        c = lax.axis_index("c")
        s = lax.axis_index("s")
        wid = c * NS + s

        pltpu.sync_copy(src_hbm.at[wid], sidx)
        pltpu.sync_copy(dst_hbm.at[wid], didx)

        @pl.loop(0, CH, step=16)
        def _(i):
            ones[pl.ds(i, 16)] = jnp.ones((16,), _f32)

        @pl.loop(0, RPT, step=16)
        def _(i):
            zb[pl.ds(i, 16)] = jnp.zeros((16,), _f32)

        pltpu.sync_copy(zb, dn_sh.at[pl.ds(s * RPT, RPT)])
        pltpu.sync_copy(zb, dh_sh.at[pl.ds(s * RPT, RPT)])
        plsc.subcore_barrier()

        @pl.loop(0, NBLK)
        def _(b):
            @pl.loop(0, SB)
            def _(j):
                pltpu.sync_copy(ones, dn_sh.at[sidx.at[b].at[j]], add=True)
                pltpu.sync_copy(ones, dh_sh.at[didx.at[b].at[j]], add=True)

        plsc.subcore_barrier()
        pltpu.sync_copy(dn_sh.at[pl.ds(s * RPT, RPT)],
                        outn_hbm.at[c].at[pl.ds(s * RPT, RPT)])
        pltpu.sync_copy(dh_sh.at[pl.ds(s * RPT, RPT)],
                        outh_hbm.at[c].at[pl.ds(s * RPT, RPT)])

    return deg


def _spmm(x, src3, dst3):
    return _spmm_sc()(x, src3, dst3)


def _deg(src3, dst3):
    return _deg_sc()(src3, dst3)


_RB = 2000


def _dot(a, b):
    return lax.dot_general(a, b, (((1,), (0,)), ((), ())),
                           precision=lax.Precision.HIGHEST,
                           preferred_element_type=_f32)


def _mm(x, w):
    def body(x_ref, w_ref, o_ref):
        o_ref[...] = _dot(x_ref[...], w_ref[...])

    return pl.pallas_call(
        body,
        grid=(N // _RB,),
        in_specs=[pl.BlockSpec((_RB, HID), lambda i: (i, 0)),
                  pl.BlockSpec((HID, HID), lambda i: (0, 0))],
        out_specs=pl.BlockSpec((_RB, HID), lambda i: (i, 0)),
        out_shape=jax.ShapeDtypeStruct((N, HID), _f32),
    )(x, w)


def _inv(dn0, dn1, dh0, dh1):
    def body(a_ref, b_ref, c_ref, d_ref, o1_ref, o2_ref):
        dn = a_ref[...] + b_ref[...]
        ri = lax.broadcasted_iota(jnp.int32, (N, 1), 0)
        dn = jnp.where(ri == 0, dn - float(EPAD - E), dn)
        dh = c_ref[...] + d_ref[...]
        o1_ref[...] = jnp.where(dn > 0, 1.0 / jnp.where(dn > 0, dn, 1.0), 0.0)
        o2_ref[...] = jnp.where(dh > 0, 1.0 / jnp.where(dh > 0, dh, 1.0), 0.0)

    return pl.pallas_call(
        body,
        grid=(1,),
        in_specs=[pl.BlockSpec((N, 1), lambda i: (0, 0))] * 4,
        out_specs=[pl.BlockSpec((N, 1), lambda i: (0, 0))] * 2,
        out_shape=[jax.ShapeDtypeStruct((N, 1), _f32)] * 2,
    )(dn0, dn1, dh0, dh1)


def _pp(i):
    return (0, i, 0)


def _pq(i):
    return (1, i, 0)


_PSPEC = [pl.BlockSpec((1, _RB, HID), _pp), pl.BlockSpec((1, _RB, HID), _pq)]


def _mid(p, binv):
    def body(p0_ref, p1_ref, s_ref, o_ref):
        o_ref[...] = s_ref[...] * (p0_ref[0] + p1_ref[0])

    return pl.pallas_call(
        body,
        grid=(N // _RB,),
        in_specs=_PSPEC + [pl.BlockSpec((_RB, 1), lambda i: (i, 0))],
        out_specs=pl.BlockSpec((_RB, HID), lambda i: (i, 0)),
        out_shape=jax.ShapeDtypeStruct((N, HID), _f32),
    )(p, p, binv)


def _endmm(q, dinv, b, w):
    def body(q0_ref, q1_ref, s_ref, b_ref, w_ref, o_ref):
        h = jnp.maximum(s_ref[...] * (q0_ref[0] + q1_ref[0]) + b_ref[...], 0.0)
        o_ref[...] = _dot(h, w_ref[...])

    return pl.pallas_call(
        body,
        grid=(N // _RB,),
        in_specs=_PSPEC + [pl.BlockSpec((_RB, 1), lambda i: (i, 0)),
                           pl.BlockSpec((1, HID), lambda i: (0, 0)),
                           pl.BlockSpec((HID, HID), lambda i: (0, 0))],
        out_specs=pl.BlockSpec((_RB, HID), lambda i: (i, 0)),
        out_shape=jax.ShapeDtypeStruct((N, HID), _f32),
    )(q, q, dinv, b, w)


def _end(q, dinv, b):
    def body(q0_ref, q1_ref, s_ref, b_ref, o_ref):
        o_ref[...] = jnp.maximum(
            s_ref[...] * (q0_ref[0] + q1_ref[0]) + b_ref[...], 0.0)

    return pl.pallas_call(
        body,
        grid=(N // _RB,),
        in_specs=_PSPEC + [pl.BlockSpec((_RB, 1), lambda i: (i, 0)),
                           pl.BlockSpec((1, HID), lambda i: (0, 0))],
        out_specs=pl.BlockSpec((_RB, HID), lambda i: (i, 0)),
        out_shape=jax.ShapeDtypeStruct((N, HID), _f32),
    )(q, q, dinv, b)


_NEG = -1e30


def _pool(h3, bm, g1w, g1b, g2w, g2b, wc, bc):
    nblk = N // _RB

    def body(h_ref, bm_ref, g1w_ref, g1b_ref, g2w_ref, g2b_ref,
             wc_ref, bc_ref, o_ref, gmax_s, num_s, den_s):
        p = pl.program_id(0)
        i = pl.program_id(1)

        @pl.when(jnp.logical_and(p == 0, i == 0))
        def _():
            gmax_s[...] = jnp.full((1, NGRAPH), _NEG, _f32)
            num_s[...] = jnp.zeros((NGRAPH, HID), _f32)
            den_s[...] = jnp.zeros((NGRAPH, 1), _f32)

        ids = lax.broadcasted_iota(jnp.int32, (1, NGRAPH), 1)
        mask = bm_ref[...] == ids
        maskf = mask.astype(_f32)
        z = jnp.maximum(_dot(h_ref[...], g1w_ref[...]) + g1b_ref[...], 0.0)
        gate = _dot(z, g2w_ref[...]) + g2b_ref[...]

        @pl.when(p == 0)
        def _():
            gm = jnp.max(jnp.where(mask, gate, _NEG), axis=0, keepdims=True)
            gmax_s[...] = jnp.maximum(gmax_s[...], gm)

        @pl.when(p == 1)
        def _():
            gmax_node = jnp.sum(jnp.where(mask, gmax_s[...], 0.0),
                                axis=1, keepdims=True)
            egate = jnp.exp(gate - gmax_node)
            den_s[...] += lax.dot_general(
                maskf, egate, (((0,), (0,)), ((), ())),
                precision=lax.Precision.HIGHEST, preferred_element_type=_f32)
            num_s[...] += lax.dot_general(
                maskf, egate * h_ref[...], (((0,), (0,)), ((), ())),
                precision=lax.Precision.HIGHEST, preferred_element_type=_f32)

            @pl.when(i == nblk - 1)
            def _():
                pooled = num_s[...] * (1.0 / (den_s[...] + 1e-16))
                o_ref[...] = _dot(pooled, wc_ref[...]) + bc_ref[...]

    return pl.pallas_call(
        body,
        grid=(2, nblk),
        in_specs=[pl.BlockSpec((_RB, HID), lambda p, i: (i, 0)),
                  pl.BlockSpec((_RB, 1), lambda p, i: (i, 0)),
                  pl.BlockSpec((HID, HID), lambda p, i: (0, 0)),
                  pl.BlockSpec((1, HID), lambda p, i: (0, 0)),
                  pl.BlockSpec((HID, 1), lambda p, i: (0, 0)),
                  pl.BlockSpec((1, 1), lambda p, i: (0, 0)),
                  pl.BlockSpec((HID, NCLS), lambda p, i: (0, 0)),
                  pl.BlockSpec((1, NCLS), lambda p, i: (0, 0))],
        out_specs=pl.BlockSpec((NGRAPH, NCLS), lambda p, i: (0, 0)),
        out_shape=jax.ShapeDtypeStruct((NGRAPH, NCLS), _f32),
        scratch_shapes=[pltpu.VMEM((1, NGRAPH), _f32),
                        pltpu.VMEM((NGRAPH, HID), _f32),
                        pltpu.VMEM((NGRAPH, 1), _f32)],
    )(h3, bm, g1w, g1b, g2w, g2b, wc, bc)


def kernel(x, edge_index, batch_map, W1, b1, W2, b2, W3, b3,
           G1, g1, G2, g2, Wc, bc):
    npad = EPAD - E
    zpad = jnp.zeros((npad,), jnp.int32)
    gpad = jnp.full((npad,), N, jnp.int32)
    sh4 = (NW, NBLK, SB, CH)
    src3 = jnp.concatenate([edge_index[0], zpad]).reshape(sh4)
    dst3 = jnp.concatenate([edge_index[1], gpad]).reshape(sh4)
    rsrc3 = jnp.concatenate([edge_index[1], zpad]).reshape(sh4)
    rdst3 = jnp.concatenate([edge_index[0], gpad]).reshape(sh4)

    degn, degh = _deg(src3, dst3)
    dinv, binv = _inv(degn[0, :N, None], degn[1, :N, None],
                      degh[0, :N, None], degh[1, :N, None])

    h = _mm(x, W1)
    for b, wnext in ((b1, W2), (b2, W3)):
        m = _mid(_spmm(h, src3, dst3), binv)
        h = _endmm(_spmm(m, rsrc3, rdst3), dinv, b[None, :], wnext)
    m = _mid(_spmm(h, src3, dst3), binv)
    h3 = _end(_spmm(m, rsrc3, rdst3), dinv, b3[None, :])

    return _pool(h3, batch_map[:, None], G1, g1[None, :], G2,
                 g2.reshape(1, 1), Wc, bc[None, :])
